# baseline (speedup 1.0000x reference)
import functools
import math

import jax
import jax.numpy as jnp
from jax import lax
from jax.experimental import pallas as pl
from jax.experimental.pallas import tpu as pltpu
from jax.experimental.pallas import tpu_sc as plsc

N = 10000
NPAD = 10240
DIN = 128
DH = 128
DOUT = 64
E = 320000
NEG = 10
TEMP = 0.07
NORM = 0.1
AUG_GAE_W = 1e-05
INS_W = 1e-05
NORM_LOSS_W = -0.1
LOGN = math.log(float(N))
DPAD = 128

BI = 1024
NB = NPAD // BI

_NC = 2
_NS = 16
_NW = _NC * _NS
_CHUNK = 128
_RPS = NPAD // _NS

_f32 = jnp.float32


@functools.lru_cache(maxsize=None)
def _make_segsum(D, ca, cb):
    mesh = plsc.VectorSubcoreMesh(core_axis_name="c", subcore_axis_name="s")

    @functools.partial(
        pl.kernel,
        out_type=jax.ShapeDtypeStruct((2 * NPAD, D), _f32),
        mesh=mesh,
        scratch_types=[
            pltpu.VMEM((_CHUNK,), jnp.int32),
            pltpu.VMEM((_CHUNK,), jnp.int32),
            pltpu.VMEM((_CHUNK, D), _f32),
            pltpu.VMEM_SHARED((NPAD, D), _f32),
            pltpu.SemaphoreType.DMA,
        ],
    )
    def seg(table, srci, dsti, zeros, out, sidx, didx, rows, acc, sem):
        cid = lax.axis_index("c")
        sid = lax.axis_index("s")
        nloc = lax.select(cid == 0, jnp.int32(ca), jnp.int32(cb))
        off = lax.select(cid == 0, sid * ca, _NS * ca + sid * cb)
        pltpu.sync_copy(zeros.at[pl.ds(sid * _RPS, _RPS)],
                        acc.at[pl.ds(sid * _RPS, _RPS)])
        plsc.subcore_barrier()

        def body(t, carry):
            base = (off + t) * _CHUNK
            pltpu.sync_copy(srci.at[pl.ds(base, _CHUNK)], sidx)
            pltpu.sync_copy(dsti.at[pl.ds(base, _CHUNK)], didx)
            pltpu.async_copy(table.at[sidx], rows, sem).wait()
            pltpu.sync_copy(rows, acc.at[didx], add=True)
            return carry

        lax.fori_loop(0, nloc, body, 0)
        plsc.subcore_barrier()
        pltpu.sync_copy(acc.at[pl.ds(sid * _RPS, _RPS)],
                        out.at[pl.ds(cid * NPAD + sid * _RPS, _RPS)])

    return seg


@functools.lru_cache(maxsize=None)
def _make_gather(D, ca, cb):
    mesh = plsc.VectorSubcoreMesh(core_axis_name="c", subcore_axis_name="s")

    @functools.partial(
        pl.kernel,
        out_type=jax.ShapeDtypeStruct((_NS * (ca + cb) * _CHUNK, D), _f32),
        mesh=mesh,
        scratch_types=[
            pltpu.VMEM((_CHUNK,), jnp.int32),
            pltpu.VMEM((_CHUNK, D), _f32),
            pltpu.SemaphoreType.DMA,
        ],
    )
    def gat(table, idx, out, iidx, rows, sem):
        cid = lax.axis_index("c")
        sid = lax.axis_index("s")
        nloc = lax.select(cid == 0, jnp.int32(ca), jnp.int32(cb))
        off = lax.select(cid == 0, sid * ca, _NS * ca + sid * cb)

        def body(t, carry):
            base = (off + t) * _CHUNK
            pltpu.sync_copy(idx.at[pl.ds(base, _CHUNK)], iidx)
            pltpu.async_copy(table.at[iidx], rows, sem).wait()
            pltpu.sync_copy(rows, out.at[pl.ds(base, _CHUNK)])
            return carry

        lax.fori_loop(0, nloc, body, 0)

    return gat


def _segsum_call(table, srci, dsti, D, split):
    zeros = jnp.zeros((NPAD, D), _f32)
    return _make_segsum(D, split[0], split[1])(table, srci, dsti, zeros)


def _gather_call(table, idx, D, split):
    return _make_gather(D, split[0], split[1])(table, idx)


def _mm_body(x_ref, w_ref, o_ref):
    o_ref[...] = jnp.dot(x_ref[...], w_ref[...],
                         preferred_element_type=_f32)


def _mm(x, w, dout):
    return pl.pallas_call(
        _mm_body,
        grid=(NB,),
        in_specs=[pl.BlockSpec((BI, x.shape[1]), lambda i: (i, 0)),
                  pl.BlockSpec((x.shape[1], dout), lambda i: (0, 0))],
        out_specs=pl.BlockSpec((BI, dout), lambda i: (i, 0)),
        out_shape=jax.ShapeDtypeStruct((NPAD, dout), _f32),
        compiler_params=pltpu.CompilerParams(
            dimension_semantics=("arbitrary",)),
    )(x, w)


def _relu_mm_body(a0_ref, a1_ref, w_ref, o_ref):
    i = pl.program_id(0)
    rows = i * BI + lax.broadcasted_iota(jnp.int32, (BI, 1), 0)
    h1 = jnp.where(rows < N, jnp.maximum(a0_ref[...] + a1_ref[...], 0.0), 0.0)
    o_ref[...] = jnp.dot(h1, w_ref[...], preferred_element_type=_f32)


def _relu_mm(acc, w, din, dout):
    return pl.pallas_call(
        _relu_mm_body,
        grid=(NB,),
        in_specs=[pl.BlockSpec((BI, din), lambda i: (i, 0)),
                  pl.BlockSpec((BI, din), lambda i: (i + NB, 0)),
                  pl.BlockSpec((din, dout), lambda i: (0, 0))],
        out_specs=pl.BlockSpec((BI, dout), lambda i: (i, 0)),
        out_shape=jax.ShapeDtypeStruct((NPAD, dout), _f32),
        compiler_params=pltpu.CompilerParams(
            dimension_semantics=("arbitrary",)),
    )(acc, acc, w)


def _relu_body(a0_ref, a1_ref, o_ref):
    i = pl.program_id(0)
    rows = i * BI + lax.broadcasted_iota(jnp.int32, (BI, 1), 0)
    o_ref[...] = jnp.where(rows < N,
                           jnp.maximum(a0_ref[...] + a1_ref[...], 0.0), 0.0)


def _relu_sum(acc, d):
    return pl.pallas_call(
        _relu_body,
        grid=(NB,),
        in_specs=[pl.BlockSpec((BI, d), lambda i: (i, 0)),
                  pl.BlockSpec((BI, d), lambda i: (i + NB, 0))],
        out_specs=pl.BlockSpec((BI, d), lambda i: (i, 0)),
        out_shape=jax.ShapeDtypeStruct((NPAD, d), _f32),
        compiler_params=pltpu.CompilerParams(
            dimension_semantics=("arbitrary",)),
    )(acc, acc)


def _softplus_neg(x):
    return jnp.log1p(jnp.exp(-jnp.abs(x))) + jnp.maximum(-x, 0.0)


def _pass1_body(hi_ref, hj_ref, m_ref, z_ref, pmi_ref, ss_ref, sg_ref):
    i = pl.program_id(0)
    j = pl.program_id(1)
    hi = hi_ref[...]
    hj = hj_ref[...]
    G = lax.dot_general(hi, hj, (((1,), (1,)), ((), ())),
                        preferred_element_type=_f32)
    rowv = (i * BI + lax.broadcasted_iota(jnp.int32, (BI, 1), 0)) < N
    colv = (j * BI + lax.broadcasted_iota(jnp.int32, (1, BI), 1)) < N
    v = jnp.logical_and(rowv, colv)

    t_ss = jnp.sum(jnp.where(v, _softplus_neg(G), 0.0))
    t_sg = jnp.sum(jnp.where(v, G, 0.0))
    sig = jax.nn.sigmoid(G)
    t_z = jnp.sum(jnp.where(colv, jnp.exp(sig), 0.0), axis=1, keepdims=True)
    t_m = jnp.max(jnp.where(colv, G, -jnp.inf), axis=1, keepdims=True)

    @pl.when(jnp.logical_and(i == 0, j == 0))
    def _():
        ss_ref[...] = jnp.zeros_like(ss_ref)
        sg_ref[...] = jnp.zeros_like(sg_ref)

    @pl.when(j == 0)
    def _():
        m_ref[...] = jnp.full_like(m_ref, -jnp.inf)
        z_ref[...] = jnp.zeros_like(z_ref)

    m_new = jnp.maximum(m_ref[...], t_m)
    z_new = z_ref[...] + t_z
    m_ref[...] = m_new
    z_ref[...] = z_new
    ss_ref[...] = ss_ref[...] + t_ss
    sg_ref[...] = sg_ref[...] + t_sg

    @pl.when(j == NB - 1)
    def _():
        pmi_ref[...] = jnp.maximum(
            jax.nn.sigmoid(m_new) - jnp.log(z_new) + LOGN, 0.0)


def _pass1(h):
    specs_row = pl.BlockSpec((BI, 1), lambda i, j: (i, 0))
    spec_scalar = pl.BlockSpec((1, 1), lambda i, j: (0, 0))
    return pl.pallas_call(
        _pass1_body,
        grid=(NB, NB),
        in_specs=[pl.BlockSpec((BI, DPAD), lambda i, j: (i, 0)),
                  pl.BlockSpec((BI, DPAD), lambda i, j: (j, 0))],
        out_specs=[specs_row, specs_row, specs_row, spec_scalar, spec_scalar],
        out_shape=[jax.ShapeDtypeStruct((NPAD, 1), _f32),
                   jax.ShapeDtypeStruct((NPAD, 1), _f32),
                   jax.ShapeDtypeStruct((NPAD, 1), _f32),
                   jax.ShapeDtypeStruct((1, 1), _f32),
                   jax.ShapeDtypeStruct((1, 1), _f32)],
        compiler_params=pltpu.CompilerParams(
            dimension_semantics=("arbitrary", "arbitrary")),
    )(h, h)


def _pass2_body(hi_ref, hj_ref, gd_ref, rad_ref, s0_ref, s1_ref, ng_ref,
                ss2_ref, sa_ref, l1_ref, l2_ref, ip_ref, ineg_ref):
    i = pl.program_id(0)
    j = pl.program_id(1)
    hi = hi_ref[...]
    gdi = gd_ref[...]
    nrm = jnp.sqrt(jnp.sum(gdi * gdi, axis=1, keepdims=True))
    gdn = gdi / jnp.maximum(nrm, 1e-12)
    aug = hi + gdn * rad_ref[...]
    hj = hj_ref[...]
    A = lax.dot_general(aug, hj, (((1,), (1,)), ((), ())),
                        preferred_element_type=_f32)
    rowv = (i * BI + lax.broadcasted_iota(jnp.int32, (BI, 1), 0)) < N
    colv = (j * BI + lax.broadcasted_iota(jnp.int32, (1, BI), 1)) < N
    v = jnp.logical_and(rowv, colv)
    t_ss2 = jnp.sum(jnp.where(v, _softplus_neg(A), 0.0))
    t_sa = jnp.sum(jnp.where(v, A, 0.0))

    @pl.when(jnp.logical_and(i == 0, j == 0))
    def _():
        ss2_ref[...] = jnp.zeros_like(ss2_ref)
        sa_ref[...] = jnp.zeros_like(sa_ref)
        l1_ref[...] = jnp.zeros_like(l1_ref)
        l2_ref[...] = jnp.zeros_like(l2_ref)
        ip_ref[...] = jnp.zeros_like(ip_ref)
        ineg_ref[...] = jnp.zeros_like(ineg_ref)

    ss2_ref[...] = ss2_ref[...] + t_ss2
    sa_ref[...] = sa_ref[...] + t_sa

    @pl.when(j == 0)
    def _():
        s_i = s0_ref[...] + s1_ref[...]
        l1_ref[...] = l1_ref[...] + jnp.sum(hi * s_i)
        l2_ref[...] = l2_ref[...] + jnp.sum(aug * s_i)
        pos = jnp.sum(aug * hi, axis=1, keepdims=True) / TEMP
        ip_ref[...] = ip_ref[...] + jnp.sum(
            jnp.where(rowv, _softplus_neg(pos), 0.0))
        ng = ng_ref[...]
        acc = jnp.zeros((), _f32)
        for k in range(NEG):
            nk = ng[:, k * DPAD:(k + 1) * DPAD]
            d = jnp.sum(aug * nk, axis=1, keepdims=True) / TEMP
            acc = acc + jnp.sum(jnp.where(rowv, _softplus_neg(-d), 0.0))
        ineg_ref[...] = ineg_ref[...] + acc


def _pass2(h, gd, rad, s, negh):
    spec_scalar = pl.BlockSpec((1, 1), lambda i, j: (0, 0))
    return pl.pallas_call(
        _pass2_body,
        grid=(NB, NB),
        in_specs=[pl.BlockSpec((BI, DPAD), lambda i, j: (i, 0)),
                  pl.BlockSpec((BI, DPAD), lambda i, j: (j, 0)),
                  pl.BlockSpec((BI, DPAD), lambda i, j: (i, 0)),
                  pl.BlockSpec((BI, 1), lambda i, j: (i, 0)),
                  pl.BlockSpec((BI, DPAD), lambda i, j: (i, 0)),
                  pl.BlockSpec((BI, DPAD), lambda i, j: (i + NB, 0)),
                  pl.BlockSpec((BI, NEG * DPAD), lambda i, j: (i, 0))],
        out_specs=[spec_scalar] * 6,
        out_shape=[jax.ShapeDtypeStruct((1, 1), _f32)] * 6,
        compiler_params=pltpu.CompilerParams(
            dimension_semantics=("arbitrary", "arbitrary")),
    )(h, h, gd, rad, s, s, negh)


def kernel(x, edge_index, adj_orig_index, gradint_dir, negative_index, W1, W2):
    src = edge_index[0]
    dst = edge_index[1]
    ai = adj_orig_index[0]
    aj = adj_orig_index[1]

    frac0 = 0.55

    def _split(n):
        tot = -(-n // _CHUNK)
        ca = -(-int(tot * frac0) // _NS)
        cb = max(1, -(-(tot - _NS * ca) // _NS))
        return ca, cb

    def _prep_idx(v, split, fill):
        ca, cb = split
        pad = _NS * (ca + cb) * _CHUNK - v.shape[0]
        return jnp.concatenate([v, jnp.full((pad,), fill, jnp.int32)])

    esp = _split(E)
    src_p = _prep_idx(src, esp, 0)
    dst_p = _prep_idx(dst, esp, N)
    ai_p = _prep_idx(ai, esp, N)
    aj_p = _prep_idx(aj, esp, 0)

    nsp = _split(N * NEG)
    neg_p = _prep_idx(negative_index.reshape(-1), nsp, 0)

    xpad = jnp.pad(x, ((0, NPAD - N), (0, 0)))
    gdpad = jnp.pad(gradint_dir, ((0, NPAD - N), (0, DPAD - DOUT)))
    w2pad = jnp.pad(W2, ((0, 0), (0, DPAD - DOUT)))

    xw1 = _mm(xpad, W1, DH)
    acc1 = _segsum_call(xw1, src_p, dst_p, DH, esp)
    xw2 = _relu_mm(acc1, w2pad, DH, DPAD)
    acc2 = _segsum_call(xw2, src_p, dst_p, DPAD, esp)
    h = _relu_sum(acc2, DPAD)

    s = _segsum_call(h, aj_p, ai_p, DPAD, esp)
    negh = _gather_call(h, neg_p, DPAD, nsp)
    negh = negh[:N * NEG].reshape(N, NEG * DPAD)
    negh = jnp.pad(negh, ((0, NPAD - N), (0, 0)))

    _, _, pmi, ss1, sg = _pass1(h)
    pmi_n = pmi[:N, 0]
    big_p = jnp.max(pmi_n)
    radius = 1.0 - pmi_n / big_p
    rad = jnp.pad(radius[:, None], ((0, NPAD - N), (0, 0)))

    ss2, sa, l1, l2, ip, ineg = _pass2(h, gdpad, rad, s, negh)

    n2 = float(N) * float(N)
    gae = NORM * (ss1[0, 0] + sg[0, 0] - l1[0, 0]) / n2
    aug_gae = NORM * (ss2[0, 0] + sa[0, 0] - l2[0, 0]) / n2 * AUG_GAE_W
    ins = (ip[0, 0] + ineg[0, 0]) / float(N) * INS_W
    norm_loss = jnp.mean(pmi_n / big_p) * NORM_LOSS_W
    return gae + aug_gae + ins + norm_loss


# speedup vs baseline: 1.0078x; 1.0078x over previous
import functools
import math

import jax
import jax.numpy as jnp
from jax import lax
from jax.experimental import pallas as pl
from jax.experimental.pallas import tpu as pltpu
from jax.experimental.pallas import tpu_sc as plsc

N = 10000
NPAD = 10240
DIN = 128
DH = 128
DOUT = 64
E = 320000
NEG = 10
TEMP = 0.07
NORM = 0.1
AUG_GAE_W = 1e-05
INS_W = 1e-05
NORM_LOSS_W = -0.1
LOGN = math.log(float(N))
DPAD = 128

BI = 1024
NB = NPAD // BI

_NC = 2
_NS = 16
_NW = _NC * _NS
_CHUNK = 128
_RPS = NPAD // _NS

_f32 = jnp.float32


@functools.lru_cache(maxsize=None)
def _make_segsum(D, ca, cb):
    mesh = plsc.VectorSubcoreMesh(core_axis_name="c", subcore_axis_name="s")

    @functools.partial(
        pl.kernel,
        out_type=jax.ShapeDtypeStruct((2 * NPAD, D), _f32),
        mesh=mesh,
        scratch_types=[
            pltpu.VMEM((_CHUNK,), jnp.int32),
            pltpu.VMEM((_CHUNK,), jnp.int32),
            pltpu.VMEM((_CHUNK, D), _f32),
            pltpu.VMEM_SHARED((NPAD, D), _f32),
            pltpu.SemaphoreType.DMA,
        ],
    )
    def seg(table, srci, dsti, zeros, out, sidx, didx, rows, acc, sem):
        cid = lax.axis_index("c")
        sid = lax.axis_index("s")
        nloc = lax.select(cid == 0, jnp.int32(ca), jnp.int32(cb))
        off = lax.select(cid == 0, sid * ca, _NS * ca + sid * cb)
        pltpu.sync_copy(zeros.at[pl.ds(sid * _RPS, _RPS)],
                        acc.at[pl.ds(sid * _RPS, _RPS)])
        plsc.subcore_barrier()

        def body(t, carry):
            base = (off + t) * _CHUNK
            pltpu.sync_copy(srci.at[pl.ds(base, _CHUNK)], sidx)
            pltpu.sync_copy(dsti.at[pl.ds(base, _CHUNK)], didx)
            pltpu.async_copy(table.at[sidx], rows, sem).wait()
            pltpu.sync_copy(rows, acc.at[didx], add=True)
            return carry

        lax.fori_loop(0, nloc, body, 0)
        plsc.subcore_barrier()
        pltpu.sync_copy(acc.at[pl.ds(sid * _RPS, _RPS)],
                        out.at[pl.ds(cid * NPAD + sid * _RPS, _RPS)])

    return seg


@functools.lru_cache(maxsize=None)
def _make_gather(D, ca, cb):
    mesh = plsc.VectorSubcoreMesh(core_axis_name="c", subcore_axis_name="s")

    @functools.partial(
        pl.kernel,
        out_type=jax.ShapeDtypeStruct((_NS * (ca + cb) * _CHUNK, D), _f32),
        mesh=mesh,
        scratch_types=[
            pltpu.VMEM((_CHUNK,), jnp.int32),
            pltpu.VMEM((_CHUNK, D), _f32),
            pltpu.SemaphoreType.DMA,
        ],
    )
    def gat(table, idx, out, iidx, rows, sem):
        cid = lax.axis_index("c")
        sid = lax.axis_index("s")
        nloc = lax.select(cid == 0, jnp.int32(ca), jnp.int32(cb))
        off = lax.select(cid == 0, sid * ca, _NS * ca + sid * cb)

        def body(t, carry):
            base = (off + t) * _CHUNK
            pltpu.sync_copy(idx.at[pl.ds(base, _CHUNK)], iidx)
            pltpu.async_copy(table.at[iidx], rows, sem).wait()
            pltpu.sync_copy(rows, out.at[pl.ds(base, _CHUNK)])
            return carry

        lax.fori_loop(0, nloc, body, 0)

    return gat


def _segsum_call(table, srci, dsti, D, split):
    zeros = jnp.zeros((NPAD, D), _f32)
    return _make_segsum(D, split[0], split[1])(table, srci, dsti, zeros)


def _gather_call(table, idx, D, split):
    return _make_gather(D, split[0], split[1])(table, idx)


def _mm_body(x_ref, w_ref, o_ref):
    o_ref[...] = jnp.dot(x_ref[...], w_ref[...],
                         preferred_element_type=_f32)


def _mm(x, w, dout):
    return pl.pallas_call(
        _mm_body,
        grid=(NB,),
        in_specs=[pl.BlockSpec((BI, x.shape[1]), lambda i: (i, 0)),
                  pl.BlockSpec((x.shape[1], dout), lambda i: (0, 0))],
        out_specs=pl.BlockSpec((BI, dout), lambda i: (i, 0)),
        out_shape=jax.ShapeDtypeStruct((NPAD, dout), _f32),
        compiler_params=pltpu.CompilerParams(
            dimension_semantics=("arbitrary",)),
    )(x, w)


def _relu_mm_body(a0_ref, a1_ref, w_ref, o_ref):
    i = pl.program_id(0)
    rows = i * BI + lax.broadcasted_iota(jnp.int32, (BI, 1), 0)
    h1 = jnp.where(rows < N, jnp.maximum(a0_ref[...] + a1_ref[...], 0.0), 0.0)
    o_ref[...] = jnp.dot(h1, w_ref[...], preferred_element_type=_f32)


def _relu_mm(acc, w, din, dout):
    return pl.pallas_call(
        _relu_mm_body,
        grid=(NB,),
        in_specs=[pl.BlockSpec((BI, din), lambda i: (i, 0)),
                  pl.BlockSpec((BI, din), lambda i: (i + NB, 0)),
                  pl.BlockSpec((din, dout), lambda i: (0, 0))],
        out_specs=pl.BlockSpec((BI, dout), lambda i: (i, 0)),
        out_shape=jax.ShapeDtypeStruct((NPAD, dout), _f32),
        compiler_params=pltpu.CompilerParams(
            dimension_semantics=("arbitrary",)),
    )(acc, acc, w)


def _relu_body(a0_ref, a1_ref, o_ref):
    i = pl.program_id(0)
    rows = i * BI + lax.broadcasted_iota(jnp.int32, (BI, 1), 0)
    o_ref[...] = jnp.where(rows < N,
                           jnp.maximum(a0_ref[...] + a1_ref[...], 0.0), 0.0)


def _relu_sum(acc, d):
    return pl.pallas_call(
        _relu_body,
        grid=(NB,),
        in_specs=[pl.BlockSpec((BI, d), lambda i: (i, 0)),
                  pl.BlockSpec((BI, d), lambda i: (i + NB, 0))],
        out_specs=pl.BlockSpec((BI, d), lambda i: (i, 0)),
        out_shape=jax.ShapeDtypeStruct((NPAD, d), _f32),
        compiler_params=pltpu.CompilerParams(
            dimension_semantics=("arbitrary",)),
    )(acc, acc)


def _softplus_neg(x):
    return jnp.log1p(jnp.exp(-jnp.abs(x))) + jnp.maximum(-x, 0.0)


def _pass1_body(hi_ref, hj_ref, m_ref, z_ref, pmi_ref, ss_ref, sg_ref):
    i = pl.program_id(0)
    j = pl.program_id(1)
    hi = hi_ref[...]
    hj = hj_ref[...]
    G = lax.dot_general(hi, hj, (((1,), (1,)), ((), ())),
                        preferred_element_type=_f32)
    rowv = (i * BI + lax.broadcasted_iota(jnp.int32, (BI, 1), 0)) < N
    colv = (j * BI + lax.broadcasted_iota(jnp.int32, (1, BI), 1)) < N
    v = jnp.logical_and(rowv, colv)

    t_ss = jnp.sum(jnp.where(v, _softplus_neg(G), 0.0))
    t_sg = jnp.sum(jnp.where(v, G, 0.0))
    sig = jax.nn.sigmoid(G)
    t_z = jnp.sum(jnp.where(colv, jnp.exp(sig), 0.0), axis=1, keepdims=True)
    t_m = jnp.max(jnp.where(colv, G, -jnp.inf), axis=1, keepdims=True)

    @pl.when(jnp.logical_and(i == 0, j == 0))
    def _():
        ss_ref[...] = jnp.zeros_like(ss_ref)
        sg_ref[...] = jnp.zeros_like(sg_ref)

    @pl.when(j == 0)
    def _():
        m_ref[...] = jnp.full_like(m_ref, -jnp.inf)
        z_ref[...] = jnp.zeros_like(z_ref)

    m_new = jnp.maximum(m_ref[...], t_m)
    z_new = z_ref[...] + t_z
    m_ref[...] = m_new
    z_ref[...] = z_new
    ss_ref[...] = ss_ref[...] + t_ss
    sg_ref[...] = sg_ref[...] + t_sg

    @pl.when(j == NB - 1)
    def _():
        pmi_ref[...] = jnp.maximum(
            jax.nn.sigmoid(m_new) - jnp.log(z_new) + LOGN, 0.0)


def _pass1(h):
    specs_row = pl.BlockSpec((BI, 1), lambda i, j: (i, 0))
    spec_scalar = pl.BlockSpec((1, 1), lambda i, j: (0, 0))
    return pl.pallas_call(
        _pass1_body,
        grid=(NB, NB),
        in_specs=[pl.BlockSpec((BI, DPAD), lambda i, j: (i, 0)),
                  pl.BlockSpec((BI, DPAD), lambda i, j: (j, 0))],
        out_specs=[specs_row, specs_row, specs_row, spec_scalar, spec_scalar],
        out_shape=[jax.ShapeDtypeStruct((NPAD, 1), _f32),
                   jax.ShapeDtypeStruct((NPAD, 1), _f32),
                   jax.ShapeDtypeStruct((NPAD, 1), _f32),
                   jax.ShapeDtypeStruct((1, 1), _f32),
                   jax.ShapeDtypeStruct((1, 1), _f32)],
        compiler_params=pltpu.CompilerParams(
            dimension_semantics=("arbitrary", "arbitrary")),
    )(h, h)


def _pass2_body(hi_ref, hj_ref, gd_ref, rad_ref, s0_ref, s1_ref, ng_ref,
                ss2_ref, sa_ref, l1_ref, l2_ref, ip_ref, ineg_ref):
    i = pl.program_id(0)
    j = pl.program_id(1)
    hi = hi_ref[...]
    gdi = gd_ref[...]
    nrm = jnp.sqrt(jnp.sum(gdi * gdi, axis=1, keepdims=True))
    gdn = gdi / jnp.maximum(nrm, 1e-12)
    aug = hi + gdn * rad_ref[...]
    hj = hj_ref[...]
    A = lax.dot_general(aug, hj, (((1,), (1,)), ((), ())),
                        preferred_element_type=_f32)
    rowv = (i * BI + lax.broadcasted_iota(jnp.int32, (BI, 1), 0)) < N
    colv = (j * BI + lax.broadcasted_iota(jnp.int32, (1, BI), 1)) < N
    v = jnp.logical_and(rowv, colv)
    t_ss2 = jnp.sum(jnp.where(v, _softplus_neg(A), 0.0))
    t_sa = jnp.sum(jnp.where(v, A, 0.0))

    @pl.when(jnp.logical_and(i == 0, j == 0))
    def _():
        ss2_ref[...] = jnp.zeros_like(ss2_ref)
        sa_ref[...] = jnp.zeros_like(sa_ref)
        l1_ref[...] = jnp.zeros_like(l1_ref)
        l2_ref[...] = jnp.zeros_like(l2_ref)
        ip_ref[...] = jnp.zeros_like(ip_ref)
        ineg_ref[...] = jnp.zeros_like(ineg_ref)

    ss2_ref[...] = ss2_ref[...] + t_ss2
    sa_ref[...] = sa_ref[...] + t_sa

    @pl.when(j == 0)
    def _():
        s_i = s0_ref[...] + s1_ref[...]
        l1_ref[...] = l1_ref[...] + jnp.sum(hi * s_i)
        l2_ref[...] = l2_ref[...] + jnp.sum(aug * s_i)
        pos = jnp.sum(aug * hi, axis=1, keepdims=True) / TEMP
        ip_ref[...] = ip_ref[...] + jnp.sum(
            jnp.where(rowv, _softplus_neg(pos), 0.0))
        ng = ng_ref[...]
        acc = jnp.zeros((), _f32)
        for k in range(NEG):
            nk = ng[:, k * DPAD:(k + 1) * DPAD]
            d = jnp.sum(aug * nk, axis=1, keepdims=True) / TEMP
            acc = acc + jnp.sum(jnp.where(rowv, _softplus_neg(-d), 0.0))
        ineg_ref[...] = ineg_ref[...] + acc


def _pass2(h, gd, rad, s, negh):
    spec_scalar = pl.BlockSpec((1, 1), lambda i, j: (0, 0))
    return pl.pallas_call(
        _pass2_body,
        grid=(NB, NB),
        in_specs=[pl.BlockSpec((BI, DPAD), lambda i, j: (i, 0)),
                  pl.BlockSpec((BI, DPAD), lambda i, j: (j, 0)),
                  pl.BlockSpec((BI, DPAD), lambda i, j: (i, 0)),
                  pl.BlockSpec((BI, 1), lambda i, j: (i, 0)),
                  pl.BlockSpec((BI, DPAD), lambda i, j: (i, 0)),
                  pl.BlockSpec((BI, DPAD), lambda i, j: (i + NB, 0)),
                  pl.BlockSpec((BI, NEG * DPAD), lambda i, j: (i, 0))],
        out_specs=[spec_scalar] * 6,
        out_shape=[jax.ShapeDtypeStruct((1, 1), _f32)] * 6,
        compiler_params=pltpu.CompilerParams(
            dimension_semantics=("arbitrary", "arbitrary")),
    )(h, h, gd, rad, s, s, negh)


def kernel(x, edge_index, adj_orig_index, gradint_dir, negative_index, W1, W2):
    src = edge_index[0]
    dst = edge_index[1]
    ai = adj_orig_index[0]
    aj = adj_orig_index[1]

    frac0 = 0.58

    def _split(n):
        tot = -(-n // _CHUNK)
        ca = -(-int(tot * frac0) // _NS)
        cb = max(1, -(-(tot - _NS * ca) // _NS))
        return ca, cb

    def _prep_idx(v, split, fill):
        ca, cb = split
        pad = _NS * (ca + cb) * _CHUNK - v.shape[0]
        return jnp.concatenate([v, jnp.full((pad,), fill, jnp.int32)])

    esp = _split(E)
    src_p = _prep_idx(src, esp, 0)
    dst_p = _prep_idx(dst, esp, N)
    ai_p = _prep_idx(ai, esp, N)
    aj_p = _prep_idx(aj, esp, 0)

    nsp = _split(N * NEG)
    neg_p = _prep_idx(negative_index.reshape(-1), nsp, 0)

    xpad = jnp.pad(x, ((0, NPAD - N), (0, 0)))
    gdpad = jnp.pad(gradint_dir, ((0, NPAD - N), (0, DPAD - DOUT)))
    w2pad = jnp.pad(W2, ((0, 0), (0, DPAD - DOUT)))

    xw1 = _mm(xpad, W1, DH)
    acc1 = _segsum_call(xw1, src_p, dst_p, DH, esp)
    xw2 = _relu_mm(acc1, w2pad, DH, DPAD)
    acc2 = _segsum_call(xw2, src_p, dst_p, DPAD, esp)
    h = _relu_sum(acc2, DPAD)

    s = _segsum_call(h, aj_p, ai_p, DPAD, esp)
    negh = _gather_call(h, neg_p, DPAD, nsp)
    negh = negh[:N * NEG].reshape(N, NEG * DPAD)
    negh = jnp.pad(negh, ((0, NPAD - N), (0, 0)))

    _, _, pmi, ss1, sg = _pass1(h)
    pmi_n = pmi[:N, 0]
    big_p = jnp.max(pmi_n)
    radius = 1.0 - pmi_n / big_p
    rad = jnp.pad(radius[:, None], ((0, NPAD - N), (0, 0)))

    ss2, sa, l1, l2, ip, ineg = _pass2(h, gdpad, rad, s, negh)

    n2 = float(N) * float(N)
    gae = NORM * (ss1[0, 0] + sg[0, 0] - l1[0, 0]) / n2
    aug_gae = NORM * (ss2[0, 0] + sa[0, 0] - l2[0, 0]) / n2 * AUG_GAE_W
    ins = (ip[0, 0] + ineg[0, 0]) / float(N) * INS_W
    norm_loss = jnp.mean(pmi_n / big_p) * NORM_LOSS_W
    return gae + aug_gae + ins + norm_loss
